# Initial kernel scaffold; baseline (speedup 1.0000x reference)
import jax
import jax.numpy as jnp
from jax.experimental import pallas as pl


def kernel(x, edge_index, edge_attr, u, batch, W, b):
    raise NotImplementedError("write your pallas kernel here")


# speedup vs baseline: 9.4187x; 9.4187x over previous
import functools

import jax
import jax.numpy as jnp
from jax import lax
from jax.experimental import pallas as pl
from jax.experimental.pallas import tpu as pltpu
from jax.experimental.pallas import tpu_sc as plsc

_N = 10000
_E = 320000
_B = 64
_FX = 128
_FE = 16
_FU = 128
_FUO = 128

_NC = 2
_NS = 16
_NW = _NC * _NS
_EPW = _E // _NW
_ECH = 2000
_NECH = _EPW // _ECH
_NPAD = 10240
_XPW = _NPAD // _NW
_BP = _B + 1
_CXP = 80


def _sc_body(row_h, eat_h, batch_h, x_h,
             e_out, ce_out, x_out, cx_out,
             batch_tab, row_buf, eat_buf, xb_buf, bb_buf,
             acc_e, cnt_e, acc_x, cnt_x):
    cid = lax.axis_index("c")
    sid = lax.axis_index("s")
    wid = sid * _NC + cid

    z16 = jnp.zeros((16,), jnp.float32)
    ones = jnp.ones((16,), jnp.float32)
    iota = lax.iota(jnp.int32, 16)

    def _z_e(i, c):
        acc_e[i] = z16
        return c
    lax.fori_loop(0, _B, _z_e, 0)
    for i in range(_B // 16):
        cnt_e[pl.ds(i * 16, 16)] = z16
    def _z_x(i, c):
        for k in range(_FX // 16):
            acc_x[i, pl.ds(k * 16, 16)] = z16
        return c
    lax.fori_loop(0, _BP, _z_x, 0)
    for i in range(_CXP // 16):
        cnt_x[pl.ds(i * 16, 16)] = z16

    xbase = wid * _XPW
    pltpu.sync_copy(x_h.at[pl.ds(xbase, _XPW)], xb_buf)
    pltpu.sync_copy(batch_h.at[pl.ds(xbase, _XPW)], bb_buf)

    def _cx(g, c):
        segv = bb_buf[pl.ds(g * 16, 16)]
        plsc.addupdate_scatter(cnt_x, [segv], ones)
        return c
    lax.fori_loop(0, _XPW // 16, _cx, 0)

    def _xs(g, c):
        segv = bb_buf[pl.ds(g * 16, 16)]
        for j in range(16):
            s = segv[j]
            for k in range(_FX // 16):
                vec = xb_buf[g * 16 + j, pl.ds(k * 16, 16)]
                plsc.addupdate(acc_x.at[s, pl.ds(k * 16, 16)], vec)
        return c
    lax.fori_loop(0, _XPW // 16, _xs, 0)

    pltpu.sync_copy(batch_h, batch_tab)
    ebase = wid * _EPW

    def _chunk(c, carry):
        pltpu.sync_copy(row_h.at[pl.ds(ebase + c * _ECH, _ECH)], row_buf)
        pltpu.sync_copy(eat_h.at[pl.ds(ebase + c * _ECH, _ECH)], eat_buf)

        def _eg(g, cc):
            rowv = row_buf[pl.ds(g * 16, 16)]
            segv = plsc.load_gather(batch_tab, [rowv])
            plsc.addupdate_scatter(cnt_e, [segv], ones)
            eidv = g * 16 + iota
            for f in range(_FE):
                fv = jnp.full((16,), f, jnp.int32)
                vals = plsc.load_gather(eat_buf, [eidv, fv])
                plsc.addupdate_scatter(acc_e, [segv, fv], vals)
            return cc
        lax.fori_loop(0, _ECH // 16, _eg, 0)
        return carry
    lax.fori_loop(0, _NECH, _chunk, 0)

    pltpu.sync_copy(acc_e, e_out.at[wid])
    pltpu.sync_copy(cnt_e, ce_out.at[wid])
    pltpu.sync_copy(acc_x, x_out.at[wid])
    pltpu.sync_copy(cnt_x, cx_out.at[wid])


_sc_call = functools.partial(
    pl.kernel,
    out_type=(
        jax.ShapeDtypeStruct((_NW, _B, _FE), jnp.float32),
        jax.ShapeDtypeStruct((_NW, _B), jnp.float32),
        jax.ShapeDtypeStruct((_NW, _BP, _FX), jnp.float32),
        jax.ShapeDtypeStruct((_NW, _CXP), jnp.float32),
    ),
    scratch_types=(
        pltpu.VMEM((_NPAD,), jnp.int32),
        pltpu.VMEM((_ECH,), jnp.int32),
        pltpu.VMEM((_ECH, _FE), jnp.float32),
        pltpu.VMEM((_XPW, _FX), jnp.float32),
        pltpu.VMEM((_XPW,), jnp.int32),
        pltpu.VMEM((_B, _FE), jnp.float32),
        pltpu.VMEM((_B,), jnp.float32),
        pltpu.VMEM((_BP, _FX), jnp.float32),
        pltpu.VMEM((_CXP,), jnp.float32),
    ),
    mesh=plsc.VectorSubcoreMesh(core_axis_name="c", subcore_axis_name="s"),
    compiler_params=pltpu.CompilerParams(needs_layout_passes=False,
                                         use_tc_tiling_on_sc=False),
)(_sc_body)


def _tc_body(ep_ref, ce_ref, xp_ref, cx_ref, u_ref, w1_ref, w2_ref, w3_ref,
             b_ref, o_ref):
    es = jnp.sum(ep_ref[...], axis=0)
    ec = jnp.sum(ce_ref[...], axis=0)
    xs = jnp.sum(xp_ref[...], axis=0)
    xc = jnp.sum(cx_ref[...], axis=0)
    x_agg = xs / jnp.maximum(xc, 1.0)[:, None]
    e_agg = es / jnp.maximum(ec, 1.0)[:, None]
    dn = (((1,), (0,)), ((), ()))
    acc = lax.dot_general(x_agg, w1_ref[...], dn,
                          preferred_element_type=jnp.float32)
    acc = acc + lax.dot_general(e_agg, w2_ref[...], dn,
                                preferred_element_type=jnp.float32)
    acc = acc + lax.dot_general(u_ref[...], w3_ref[...], dn,
                                preferred_element_type=jnp.float32)
    o_ref[...] = acc + b_ref[...]


def kernel(x, edge_index, edge_attr, u, batch, W, b):
    row = edge_index[0]
    x_pad = jnp.concatenate(
        [x, jnp.zeros((_NPAD - _N, _FX), x.dtype)], axis=0)
    batch_pad = jnp.concatenate(
        [batch, jnp.full((_NPAD - _N,), _B, batch.dtype)], axis=0)

    e_part, ce_p, x_part, cx_p = _sc_call(row, edge_attr, batch_pad, x_pad)

    w1 = W[:_FX]
    w2 = W[_FX:_FX + _FE]
    w3 = W[_FX + _FE:]
    b2 = b.reshape(1, _FUO)

    out = pl.pallas_call(
        _tc_body,
        out_shape=jax.ShapeDtypeStruct((_B, _FUO), jnp.float32),
    )(e_part, ce_p, x_part[:, :_B, :], cx_p[:, :_B], u, w1, w2, w3, b2)
    return out


# speedup vs baseline: 14.9072x; 1.5827x over previous
import functools

import jax
import jax.numpy as jnp
from jax import lax
from jax.experimental import pallas as pl
from jax.experimental.pallas import tpu as pltpu
from jax.experimental.pallas import tpu_sc as plsc

_N = 10000
_E = 320000
_B = 64
_FX = 128
_FE = 16
_FU = 128
_FUO = 128

_NC = 2
_NS = 16
_NW = _NC * _NS
_EPW = _E // _NW
_ECH = 2000
_NECH = _EPW // _ECH
_SEGB = 80
_NSEG = _EPW // _SEGB
_SPC = _ECH // _SEGB
_NPAD = 10240
_XPW = _NPAD // _NW
_NXB = _XPW // _SEGB
_BP = _B + 1
_CXP = 80


def _sc_body(row_h, eat_h, batch_h, x_h,
             e_out, ce_out, x_out, cx_out,
             batch_tab, row_buf, seg2, eat_buf, xb_buf, bb2,
             ze, zx, cnt_e, cnt_x, sh_e, sh_x):
    cid = lax.axis_index("c")
    sid = lax.axis_index("s")
    wid = sid * _NC + cid

    z16 = jnp.zeros((16,), jnp.float32)
    ones = jnp.ones((16,), jnp.float32)

    def _z_e(i, c):
        ze[i] = z16
        return c
    lax.fori_loop(0, _B, _z_e, 0)
    for i in range(_B // 16):
        cnt_e[pl.ds(i * 16, 16)] = z16

    def _z_x(i, c):
        for k in range(_FX // 16):
            zx[i, pl.ds(k * 16, 16)] = z16
        return c
    lax.fori_loop(0, _BP, _z_x, 0)
    for i in range(_CXP // 16):
        cnt_x[pl.ds(i * 16, 16)] = z16

    pltpu.sync_copy(ze, sh_e.at[pl.ds(sid * _B, _B)])
    pltpu.sync_copy(zx, sh_x.at[pl.ds(sid * _BP, _BP)])

    pltpu.sync_copy(batch_h, batch_tab)
    ebase = wid * _EPW
    pltpu.sync_copy(row_h.at[pl.ds(ebase, _EPW)], row_buf)
    eoff = sid * _B

    def _seg(g, c):
        rowv = row_buf[pl.ds(g * 16, 16)]
        segv = plsc.load_gather(batch_tab, [rowv])
        plsc.addupdate_scatter(cnt_e, [segv], ones)
        r = g // (_SEGB // 16)
        col = (g % (_SEGB // 16)) * 16
        seg2[r, pl.ds(col, 16)] = segv + eoff
        return c
    lax.fori_loop(0, _EPW // 16, _seg, 0)

    def _chunk(c, carry):
        pltpu.sync_copy(eat_h.at[pl.ds(ebase + c * _ECH, _ECH)], eat_buf)
        def _tr(i, cc):
            pltpu.sync_copy(eat_buf.at[pl.ds(i * _SEGB, _SEGB)],
                            sh_e.at[seg2.at[c * _SPC + i]], add=True)
            return cc
        lax.fori_loop(0, _SPC, _tr, 0)
        return carry
    lax.fori_loop(0, _NECH, _chunk, 0)

    xbase = wid * _XPW
    xoff = sid * _BP

    def _xblk(t, carry):
        pltpu.sync_copy(x_h.at[pl.ds(xbase + t * _SEGB, _SEGB)], xb_buf)
        pltpu.sync_copy(batch_h.at[pl.ds(xbase + t * _SEGB, _SEGB)],
                        bb2.at[t])
        def _cx(g, cc):
            segv = bb2[t, pl.ds(g * 16, 16)]
            plsc.addupdate_scatter(cnt_x, [segv], ones)
            bb2[t, pl.ds(g * 16, 16)] = segv + xoff
            return cc
        lax.fori_loop(0, _SEGB // 16, _cx, 0)
        pltpu.sync_copy(xb_buf, sh_x.at[bb2.at[t]], add=True)
        return carry
    lax.fori_loop(0, _NXB, _xblk, 0)

    pltpu.sync_copy(sh_e.at[pl.ds(sid * _B, _B)], e_out.at[wid])
    pltpu.sync_copy(cnt_e, ce_out.at[wid])
    pltpu.sync_copy(sh_x.at[pl.ds(sid * _BP, _BP)], x_out.at[wid])
    pltpu.sync_copy(cnt_x, cx_out.at[wid])


_sc_call = functools.partial(
    pl.kernel,
    out_type=(
        jax.ShapeDtypeStruct((_NW, _B, _FE), jnp.float32),
        jax.ShapeDtypeStruct((_NW, _B), jnp.float32),
        jax.ShapeDtypeStruct((_NW, _BP, _FX), jnp.float32),
        jax.ShapeDtypeStruct((_NW, _CXP), jnp.float32),
    ),
    scratch_types=(
        pltpu.VMEM((_NPAD,), jnp.int32),
        pltpu.VMEM((_EPW,), jnp.int32),
        pltpu.VMEM((_NSEG, _SEGB), jnp.int32),
        pltpu.VMEM((_ECH, _FE), jnp.float32),
        pltpu.VMEM((_SEGB, _FX), jnp.float32),
        pltpu.VMEM((_NXB, _SEGB), jnp.int32),
        pltpu.VMEM((_B, _FE), jnp.float32),
        pltpu.VMEM((_BP, _FX), jnp.float32),
        pltpu.VMEM((_B,), jnp.float32),
        pltpu.VMEM((_CXP,), jnp.float32),
        pltpu.VMEM_SHARED((_NS * _B, _FE), jnp.float32),
        pltpu.VMEM_SHARED((_NS * _BP, _FX), jnp.float32),
    ),
    mesh=plsc.VectorSubcoreMesh(core_axis_name="c", subcore_axis_name="s"),
    compiler_params=pltpu.CompilerParams(needs_layout_passes=False,
                                         use_tc_tiling_on_sc=False),
)(_sc_body)


def _tc_body(ep_ref, ce_ref, xp_ref, cx_ref, u_ref, w1_ref, w2_ref, w3_ref,
             b_ref, o_ref):
    es = jnp.sum(ep_ref[...], axis=0)
    ec = jnp.sum(ce_ref[...], axis=0)
    xs = jnp.sum(xp_ref[...], axis=0)
    xc = jnp.sum(cx_ref[...], axis=0)
    x_agg = xs / jnp.maximum(xc, 1.0)[:, None]
    e_agg = es / jnp.maximum(ec, 1.0)[:, None]
    dn = (((1,), (0,)), ((), ()))
    acc = lax.dot_general(x_agg, w1_ref[...], dn,
                          preferred_element_type=jnp.float32)
    acc = acc + lax.dot_general(e_agg, w2_ref[...], dn,
                                preferred_element_type=jnp.float32)
    acc = acc + lax.dot_general(u_ref[...], w3_ref[...], dn,
                                preferred_element_type=jnp.float32)
    o_ref[...] = acc + b_ref[...]


def kernel(x, edge_index, edge_attr, u, batch, W, b):
    row = edge_index[0]
    x_pad = jnp.concatenate(
        [x, jnp.zeros((_NPAD - _N, _FX), x.dtype)], axis=0)
    batch_pad = jnp.concatenate(
        [batch, jnp.full((_NPAD - _N,), _B, batch.dtype)], axis=0)

    e_part, ce_p, x_part, cx_p = _sc_call(row, edge_attr, batch_pad, x_pad)

    w1 = W[:_FX]
    w2 = W[_FX:_FX + _FE]
    w3 = W[_FX + _FE:]
    b2 = b.reshape(1, _FUO)

    out = pl.pallas_call(
        _tc_body,
        out_shape=jax.ShapeDtypeStruct((_B, _FUO), jnp.float32),
    )(e_part, ce_p, x_part[:, :_B, :], cx_p[:, :_B], u, w1, w2, w3, b2)
    return out


# speedup vs baseline: 16.2720x; 1.0916x over previous
import functools

import jax
import jax.numpy as jnp
from jax import lax
from jax.experimental import pallas as pl
from jax.experimental.pallas import tpu as pltpu
from jax.experimental.pallas import tpu_sc as plsc

_N = 10000
_E = 320000
_B = 64
_FX = 128
_FE = 16
_FU = 128
_FUO = 128

_NC = 2
_NS = 16
_NW = _NC * _NS
_EPW = _E // _NW
_ECH = 2000
_NECH = _EPW // _ECH
_SEGB = 80
_NSEG = _EPW // _SEGB
_SPC = _ECH // _SEGB
_NPAD = 10240
_XPW = _NPAD // _NW
_NXB = _XPW // _SEGB
_BP = _B + 1
_CXP = 80
_NSLOT = 4
_NPART = _NSLOT * _NC


def _sc_body(row_h, eat_h, batch_h, x_h,
             e_out, ce_out, x_out, cx_out,
             batch_tab, row_buf, seg2, eat_buf, xb_buf, bb2,
             ze, zx, cnt_e, cnt_x, sh_e, sh_x,
             sem_bt, sem_row, sem_eb0, sem_eb1, sem_ab0, sem_ab1,
             sem_xb0, sem_xb1, sem_ax0, sem_ax1, sem_bb):
    sem_eb = (sem_eb0, sem_eb1)
    sem_ab = (sem_ab0, sem_ab1)
    sem_xb = (sem_xb0, sem_xb1)
    sem_ax = (sem_ax0, sem_ax1)
    cid = lax.axis_index("c")
    sid = lax.axis_index("s")
    wid = sid * _NC + cid

    z16 = jnp.zeros((16,), jnp.float32)
    ones = jnp.ones((16,), jnp.float32)

    ebase = wid * _EPW
    xbase = wid * _XPW
    slot = sid % _NSLOT
    eoff = slot * _B
    xoff = slot * _BP

    d_bt = pltpu.async_copy(batch_h, batch_tab, sem_bt)
    d_row = pltpu.async_copy(row_h.at[pl.ds(ebase, _EPW)], row_buf, sem_row)
    d_eat = [None] * _NECH
    d_eat[0] = pltpu.async_copy(eat_h.at[pl.ds(ebase, _ECH)],
                                eat_buf.at[0], sem_eb[0])
    d_xb = [None] * _NXB
    d_xb[0] = pltpu.async_copy(x_h.at[pl.ds(xbase, _SEGB)],
                               xb_buf.at[0], sem_xb[0])
    d_bb = [pltpu.async_copy(batch_h.at[pl.ds(xbase + t * _SEGB, _SEGB)],
                             bb2.at[t], sem_bb)
            for t in range(_NXB)]

    def _z_e(i, c):
        ze[i] = z16
        return c
    lax.fori_loop(0, _B, _z_e, 0)
    for i in range(_B // 16):
        cnt_e[pl.ds(i * 16, 16)] = z16

    def _z_x(i, c):
        for k in range(_FX // 16):
            zx[i, pl.ds(k * 16, 16)] = z16
        return c
    lax.fori_loop(0, _BP, _z_x, 0)
    for i in range(_CXP // 16):
        cnt_x[pl.ds(i * 16, 16)] = z16

    @pl.when(sid < _NSLOT)
    def _zero_slot():
        pltpu.sync_copy(ze, sh_e.at[pl.ds(sid * _B, _B)])
        pltpu.sync_copy(zx, sh_x.at[pl.ds(sid * _BP, _BP)])
    plsc.subcore_barrier()

    d_bt.wait()
    d_row.wait()

    def _seg(g, c):
        rowv = row_buf[pl.ds(g * 16, 16)]
        segv = plsc.load_gather(batch_tab, [rowv])
        plsc.addupdate_scatter(cnt_e, [segv], ones)
        r = g // (_SEGB // 16)
        col = (g % (_SEGB // 16)) * 16
        seg2[r, pl.ds(col, 16)] = segv + eoff
        return c
    lax.fori_loop(0, _EPW // 16, _seg, 0)

    d_add = [[] for _ in range(_NECH)]
    for c in range(_NECH):
        d_eat[c].wait()
        if c + 1 < _NECH:
            if c >= 1:
                for d in d_add[c - 1]:
                    d.wait()
            d_eat[c + 1] = pltpu.async_copy(
                eat_h.at[pl.ds(ebase + (c + 1) * _ECH, _ECH)],
                eat_buf.at[(c + 1) % 2], sem_eb[(c + 1) % 2])
        for i in range(_SPC):
            d_add[c].append(pltpu.async_copy(
                eat_buf.at[c % 2, pl.ds(i * _SEGB, _SEGB)],
                sh_e.at[seg2.at[c * _SPC + i]], sem_ab[c % 2], add=True))

    for d in d_bb:
        d.wait()

    def _cxall(g, c):
        segv = bb2[g // (_SEGB // 16), pl.ds((g % (_SEGB // 16)) * 16, 16)]
        plsc.addupdate_scatter(cnt_x, [segv], ones)
        bb2[g // (_SEGB // 16), pl.ds((g % (_SEGB // 16)) * 16, 16)] = \
            segv + xoff
        return c
    lax.fori_loop(0, _XPW // 16, _cxall, 0)

    d_ax = [None] * _NXB
    for t in range(_NXB):
        d_xb[t].wait()
        if t + 1 < _NXB:
            if t >= 1:
                d_ax[t - 1].wait()
            d_xb[t + 1] = pltpu.async_copy(
                x_h.at[pl.ds(xbase + (t + 1) * _SEGB, _SEGB)],
                xb_buf.at[(t + 1) % 2], sem_xb[(t + 1) % 2])
        d_ax[t] = pltpu.async_copy(xb_buf.at[t % 2], sh_x.at[bb2.at[t]],
                                   sem_ax[t % 2], add=True)

    for d in d_add[_NECH - 2]:
        d.wait()
    for d in d_add[_NECH - 1]:
        d.wait()
    d_ax[_NXB - 2].wait()
    d_ax[_NXB - 1].wait()

    pltpu.sync_copy(cnt_e, ce_out.at[wid])
    pltpu.sync_copy(cnt_x, cx_out.at[wid])

    plsc.subcore_barrier()

    @pl.when(sid < _NSLOT)
    def _publish():
        p = sid * _NC + cid
        pltpu.sync_copy(sh_e.at[pl.ds(sid * _B, _B)], e_out.at[p])
        pltpu.sync_copy(sh_x.at[pl.ds(sid * _BP, _BP)], x_out.at[p])


_sc_call = functools.partial(
    pl.kernel,
    out_type=(
        jax.ShapeDtypeStruct((_NPART, _B, _FE), jnp.float32),
        jax.ShapeDtypeStruct((_NW, _B), jnp.float32),
        jax.ShapeDtypeStruct((_NPART, _BP, _FX), jnp.float32),
        jax.ShapeDtypeStruct((_NW, _CXP), jnp.float32),
    ),
    scratch_types=(
        pltpu.VMEM((_NPAD,), jnp.int32),
        pltpu.VMEM((_EPW,), jnp.int32),
        pltpu.VMEM((_NSEG, _SEGB), jnp.int32),
        pltpu.VMEM((2, _ECH, _FE), jnp.float32),
        pltpu.VMEM((2, _SEGB, _FX), jnp.float32),
        pltpu.VMEM((_NXB, _SEGB), jnp.int32),
        pltpu.VMEM((_B, _FE), jnp.float32),
        pltpu.VMEM((_BP, _FX), jnp.float32),
        pltpu.VMEM((_B,), jnp.float32),
        pltpu.VMEM((_CXP,), jnp.float32),
        pltpu.VMEM_SHARED((_NSLOT * _B, _FE), jnp.float32),
        pltpu.VMEM_SHARED((_NSLOT * _BP, _FX), jnp.float32),
    ) + (pltpu.SemaphoreType.DMA,) * 11,
    mesh=plsc.VectorSubcoreMesh(core_axis_name="c", subcore_axis_name="s"),
    compiler_params=pltpu.CompilerParams(needs_layout_passes=False,
                                         use_tc_tiling_on_sc=False),
)(_sc_body)


def _tc_body(ep_ref, ce_ref, xp_ref, cx_ref, u_ref, w1_ref, w2_ref, w3_ref,
             b_ref, o_ref):
    es = jnp.sum(ep_ref[...], axis=0)
    ec = jnp.sum(ce_ref[...], axis=0)
    xs = jnp.sum(xp_ref[...], axis=0)
    xc = jnp.sum(cx_ref[...], axis=0)
    x_agg = xs / jnp.maximum(xc, 1.0)[:, None]
    e_agg = es / jnp.maximum(ec, 1.0)[:, None]
    dn = (((1,), (0,)), ((), ()))
    acc = lax.dot_general(x_agg, w1_ref[...], dn,
                          preferred_element_type=jnp.float32)
    acc = acc + lax.dot_general(e_agg, w2_ref[...], dn,
                                preferred_element_type=jnp.float32)
    acc = acc + lax.dot_general(u_ref[...], w3_ref[...], dn,
                                preferred_element_type=jnp.float32)
    o_ref[...] = acc + b_ref[...]


def kernel(x, edge_index, edge_attr, u, batch, W, b):
    row = edge_index[0]
    x_pad = jnp.concatenate(
        [x, jnp.zeros((_NPAD - _N, _FX), x.dtype)], axis=0)
    batch_pad = jnp.concatenate(
        [batch, jnp.full((_NPAD - _N,), _B, batch.dtype)], axis=0)

    e_part, ce_p, x_part, cx_p = _sc_call(row, edge_attr, batch_pad, x_pad)

    w1 = W[:_FX]
    w2 = W[_FX:_FX + _FE]
    w3 = W[_FX + _FE:]
    b2 = b.reshape(1, _FUO)

    out = pl.pallas_call(
        _tc_body,
        out_shape=jax.ShapeDtypeStruct((_B, _FUO), jnp.float32),
    )(e_part, ce_p, x_part[:, :_B, :], cx_p[:, :_B], u, w1, w2, w3, b2)
    return out


# speedup vs baseline: 26.9182x; 1.6543x over previous
import functools

import jax
import jax.numpy as jnp
from jax import lax
from jax.experimental import pallas as pl
from jax.experimental.pallas import tpu as pltpu
from jax.experimental.pallas import tpu_sc as plsc

_N = 10000
_E = 320000
_B = 64
_FX = 128
_FE = 16
_FU = 128
_FUO = 128

_NC = 2
_NS = 16
_NW = _NC * _NS
_NBLK = _E // 128
_CBK = 16
_NCHK = 5
_SEGB = 80
_NPAD = 10240
_XPW = _NPAD // _NW
_NXB = _XPW // _SEGB
_BP = _B + 1
_CXP = 80
_NSLOT = 4
_NPART = _NSLOT * _NC


def _sc_body(eat4_h, row_h, batch_h, x_h,
             et_out, ce_out, x_out, cx_out,
             batch_tab, row_buf, ebuf, xb_buf, bb2,
             a0, a1, a2, a3, a4, a5, a6, a7,
             a8, a9, a10, a11, a12, a13, a14, a15,
             cnt_e, cnt_x, zx, etbuf, sh_x,
             sem_bt, sem_e0, sem_e1, sem_r0, sem_r1,
             sem_xb0, sem_xb1, sem_ax0, sem_ax1, sem_bb):
    accs = (a0, a1, a2, a3, a4, a5, a6, a7,
            a8, a9, a10, a11, a12, a13, a14, a15)
    sem_e = (sem_e0, sem_e1)
    sem_r = (sem_r0, sem_r1)
    sem_xb = (sem_xb0, sem_xb1)
    sem_ax = (sem_ax0, sem_ax1)

    cid = lax.axis_index("c")
    sid = lax.axis_index("s")
    wid = sid * _NC + cid

    z16 = jnp.zeros((16,), jnp.float32)
    ones = jnp.ones((16,), jnp.float32)

    slot = sid % _NSLOT
    xoff = slot * _BP
    xbase = wid * _XPW

    sblk = 78 * wid + jnp.minimum(wid, 4)
    nblk = 78 + (wid < 4).astype(jnp.int32)

    def cstart(ci):
        return jnp.minimum(sblk + _CBK * ci, _NBLK - _CBK)

    d_bt = pltpu.async_copy(batch_h, batch_tab, sem_bt)
    d_e = [None] * _NCHK
    d_r = [None] * _NCHK
    cs0 = cstart(0)
    d_e[0] = [pltpu.async_copy(eat4_h.at[fb, pl.ds(cs0, _CBK)],
                               ebuf.at[0, fb], sem_e[0]) for fb in range(2)]
    d_r[0] = pltpu.async_copy(row_h.at[pl.ds(cs0 * 128, _CBK * 128)],
                              row_buf.at[0], sem_r[0])
    d_xb = [None] * _NXB
    d_xb[0] = pltpu.async_copy(x_h.at[pl.ds(xbase, _SEGB)],
                               xb_buf.at[0], sem_xb[0])
    d_bb = [pltpu.async_copy(batch_h.at[pl.ds(xbase + t * _SEGB, _SEGB)],
                             bb2.at[t], sem_bb)
            for t in range(_NXB)]

    for f in range(_FE):
        for q in range(_B // 16):
            accs[f][pl.ds(q * 16, 16)] = z16
    for q in range(_B // 16):
        cnt_e[pl.ds(q * 16, 16)] = z16
    for q in range(_CXP // 16):
        cnt_x[pl.ds(q * 16, 16)] = z16

    def _z_x(i, c):
        for k in range(_FX // 16):
            zx[i, pl.ds(k * 16, 16)] = z16
        return c
    lax.fori_loop(0, _BP, _z_x, 0)

    @pl.when(sid < _NSLOT)
    def _zero_slot():
        pltpu.sync_copy(zx, sh_x.at[pl.ds(sid * _BP, _BP)])
    plsc.subcore_barrier()

    d_bt.wait()

    for d in d_bb:
        d.wait()

    def _cxall(g, c):
        t = g // (_SEGB // 16)
        col = (g % (_SEGB // 16)) * 16
        segv = bb2[t, pl.ds(col, 16)]
        plsc.addupdate_scatter(cnt_x, [segv], ones)
        bb2[t, pl.ds(col, 16)] = segv + xoff
        return c
    lax.fori_loop(0, _XPW // 16, _cxall, 0)

    d_ax = [None] * _NXB
    for t in range(_NXB):
        d_xb[t].wait()
        if t + 1 < _NXB:
            if t >= 1:
                d_ax[t - 1].wait()
            d_xb[t + 1] = pltpu.async_copy(
                x_h.at[pl.ds(xbase + (t + 1) * _SEGB, _SEGB)],
                xb_buf.at[(t + 1) % 2], sem_xb[(t + 1) % 2])
        d_ax[t] = pltpu.async_copy(xb_buf.at[t % 2], sh_x.at[bb2.at[t]],
                                   sem_ax[t % 2], add=True)

    for ci in range(_NCHK):
        for d in d_e[ci]:
            d.wait()
        d_r[ci].wait()
        if ci + 1 < _NCHK:
            csn = cstart(ci + 1)
            pb = (ci + 1) % 2
            d_e[ci + 1] = [pltpu.async_copy(
                eat4_h.at[fb, pl.ds(csn, _CBK)], ebuf.at[pb, fb], sem_e[pb])
                for fb in range(2)]
            d_r[ci + 1] = pltpu.async_copy(
                row_h.at[pl.ds(csn * 128, _CBK * 128)], row_buf.at[pb],
                sem_r[pb])
        lo = sblk + _CBK * ci - cstart(ci)
        hi = lo + jnp.clip(nblk - _CBK * ci, 0, _CBK)
        pb = ci % 2

        def _blk(bi, c):
            def _grp(h, cc):
                rowv = row_buf[pb, pl.ds(bi * 128 + h * 16, 16)]
                segv = plsc.load_gather(batch_tab, [rowv])
                plsc.addupdate_scatter(cnt_e, [segv], ones)
                for f in range(_FE):
                    vals = ebuf[pb, f // 8, bi, f % 8, pl.ds(h * 16, 16)]
                    plsc.addupdate_scatter(accs[f], [segv], vals)
                return cc
            lax.fori_loop(0, 8, _grp, 0)
            return c
        lax.fori_loop(lo, hi, _blk, 0)

    d_ax[_NXB - 2].wait()
    d_ax[_NXB - 1].wait()

    for f in range(_FE):
        for q in range(_B // 16):
            etbuf[f, pl.ds(q * 16, 16)] = accs[f][pl.ds(q * 16, 16)]
    pltpu.sync_copy(etbuf, et_out.at[wid])
    pltpu.sync_copy(cnt_e, ce_out.at[wid])
    pltpu.sync_copy(cnt_x, cx_out.at[wid])

    plsc.subcore_barrier()

    @pl.when(sid < _NSLOT)
    def _publish():
        p = sid * _NC + cid
        pltpu.sync_copy(sh_x.at[pl.ds(sid * _BP, _BP)], x_out.at[p])


_sc_call = functools.partial(
    pl.kernel,
    out_type=(
        jax.ShapeDtypeStruct((_NW, _FE, _B), jnp.float32),
        jax.ShapeDtypeStruct((_NW, _B), jnp.float32),
        jax.ShapeDtypeStruct((_NPART, _BP, _FX), jnp.float32),
        jax.ShapeDtypeStruct((_NW, _CXP), jnp.float32),
    ),
    scratch_types=(
        pltpu.VMEM((_NPAD,), jnp.int32),
        pltpu.VMEM((2, _CBK * 128), jnp.int32),
        pltpu.VMEM((2, 2, _CBK, 8, 128), jnp.float32),
        pltpu.VMEM((2, _SEGB, _FX), jnp.float32),
        pltpu.VMEM((_NXB, _SEGB), jnp.int32),
    ) + (pltpu.VMEM((_B,), jnp.float32),) * _FE
    + (
        pltpu.VMEM((_B,), jnp.float32),
        pltpu.VMEM((_CXP,), jnp.float32),
        pltpu.VMEM((_BP, _FX), jnp.float32),
        pltpu.VMEM((_FE, _B), jnp.float32),
        pltpu.VMEM_SHARED((_NSLOT * _BP, _FX), jnp.float32),
    ) + (pltpu.SemaphoreType.DMA,) * 10,
    mesh=plsc.VectorSubcoreMesh(core_axis_name="c", subcore_axis_name="s"),
    compiler_params=pltpu.CompilerParams(needs_layout_passes=False,
                                         use_tc_tiling_on_sc=False),
)(_sc_body)


def _tc_body(ep_ref, ce_ref, xp_ref, cx_ref, u_ref, w1_ref, w2_ref, w3_ref,
             b_ref, o_ref):
    es = jnp.sum(ep_ref[...], axis=0)
    ec = jnp.sum(ce_ref[...], axis=0)
    xs = jnp.sum(xp_ref[...], axis=0)
    xc = jnp.sum(cx_ref[...], axis=0)
    x_agg = xs / jnp.maximum(xc, 1.0)[:, None]
    e_aggt = es / jnp.maximum(ec, 1.0)[None, :]
    dn = (((1,), (0,)), ((), ()))
    dnt = (((0,), (0,)), ((), ()))
    acc = lax.dot_general(x_agg, w1_ref[...], dn,
                          preferred_element_type=jnp.float32)
    acc = acc + lax.dot_general(e_aggt, w2_ref[...], dnt,
                                preferred_element_type=jnp.float32)
    acc = acc + lax.dot_general(u_ref[...], w3_ref[...], dn,
                                preferred_element_type=jnp.float32)
    o_ref[...] = acc + b_ref[...]


def kernel(x, edge_index, edge_attr, u, batch, W, b):
    row = edge_index[0]
    eat4 = edge_attr.T.reshape(2, 8, _NBLK, 128).transpose(0, 2, 1, 3)
    x_pad = jnp.concatenate(
        [x, jnp.zeros((_NPAD - _N, _FX), x.dtype)], axis=0)
    batch_pad = jnp.concatenate(
        [batch, jnp.full((_NPAD - _N,), _B, batch.dtype)], axis=0)

    e_part, ce_p, x_part, cx_p = _sc_call(eat4, row, batch_pad, x_pad)

    w1 = W[:_FX]
    w2 = W[_FX:_FX + _FE]
    w3 = W[_FX + _FE:]
    b2 = b.reshape(1, _FUO)

    out = pl.pallas_call(
        _tc_body,
        out_shape=jax.ShapeDtypeStruct((_B, _FUO), jnp.float32),
    )(e_part, ce_p, x_part[:, :_B, :], cx_p[:, :_B], u, w1, w2, w3, b2)
    return out
